# baseline (speedup 1.0000x reference)
import jax
import jax.numpy as jnp
from jax.experimental import pallas as pl
from jax.experimental.pallas import tpu as pltpu

INPUT_DIM = 1024
LATENT_DIM = 4096
K = 64
TM = 256
INT_MIN = -(2**31)


def _count(mask):
    return jnp.sum(mask.astype(jnp.int32), axis=1, keepdims=True)


def _make_body(nstep):
    def _body(x_ref, we_ref, be_ref, wd_ref, bd_ref, sp_ref, rec_ref, lat_sc):
        i = pl.program_id(0)

        @pl.when(i < nstep)
        def _encode():
            lat_sc[i % 2] = jax.lax.dot_general(
                x_ref[...], we_ref[...], (((1,), (1,)), ((), ())),
                preferred_element_type=jnp.float32,
            ) + be_ref[...]

        @pl.when(i > 0)
        def _select_decode():
            lat = lat_sc[(i - 1) % 2]

            ikey = jax.lax.bitcast_convert_type(lat, jnp.int32)
            key = jnp.where(ikey < 0, ikey ^ jnp.int32(0x7FFFFFFF), ikey)

            t = jnp.where(_count(key >= 0) >= K,
                          jnp.int32(0), jnp.int32(INT_MIN))
            for bit in range(30, -1, -1):
                cand = t + jnp.int32(1 << bit)
                t = jnp.where(_count(key >= cand) >= K, cand, t)

            sparse = jnp.where(key >= t, lat, 0.0)
            sp_ref[...] = sparse

            rec = jax.lax.dot_general(
                sparse.astype(jnp.bfloat16), wd_ref[...],
                (((1,), (1,)), ((), ())),
                preferred_element_type=jnp.float32,
            ) + bd_ref[...]
            rec_ref[...] = rec

    return _body


@jax.jit
def kernel(x, W_enc, b_enc, W_dec, b_dec):
    B = x.shape[0]
    nstep = B // TM
    out = pl.pallas_call(
        _make_body(nstep),
        grid=(nstep + 1,),
        in_specs=[
            pl.BlockSpec((TM, INPUT_DIM), lambda i: (jnp.minimum(i, nstep - 1), 0)),
            pl.BlockSpec((LATENT_DIM, INPUT_DIM), lambda i: (0, 0)),
            pl.BlockSpec((1, LATENT_DIM), lambda i: (0, 0)),
            pl.BlockSpec((INPUT_DIM, LATENT_DIM), lambda i: (0, 0)),
            pl.BlockSpec((1, INPUT_DIM), lambda i: (0, 0)),
        ],
        out_specs=[
            pl.BlockSpec((TM, LATENT_DIM), lambda i: (jnp.maximum(i - 1, 0), 0)),
            pl.BlockSpec((TM, INPUT_DIM), lambda i: (jnp.maximum(i - 1, 0), 0)),
        ],
        out_shape=[
            jax.ShapeDtypeStruct((B, LATENT_DIM), jnp.float32),
            jax.ShapeDtypeStruct((B, INPUT_DIM), jnp.float32),
        ],
        scratch_shapes=[pltpu.VMEM((2, TM, LATENT_DIM), jnp.float32)],
        compiler_params=pltpu.CompilerParams(
            vmem_limit_bytes=100 * 1024 * 1024,
        ),
    )(x, W_enc, b_enc.reshape(1, LATENT_DIM),
      W_dec.astype(jnp.bfloat16), b_dec.reshape(1, INPUT_DIM))
    sparse, recon = out
    return (recon, sparse)


# speedup vs baseline: 1.0497x; 1.0497x over previous
import jax
import jax.numpy as jnp
from jax.experimental import pallas as pl
from jax.experimental.pallas import tpu as pltpu

INPUT_DIM = 1024
LATENT_DIM = 4096
K = 64
TM = 256
INT_MIN = -(2**31)


def _count(mask):
    return jnp.sum(mask.astype(jnp.int32), axis=1, keepdims=True)


def _body(x_ref, we_ref, be_ref, wd_ref, bd_ref, sp_ref, rec_ref):
    lat = jax.lax.dot_general(
        x_ref[...], we_ref[...], (((1,), (1,)), ((), ())),
        preferred_element_type=jnp.float32,
    ) + be_ref[...]

    ikey = jax.lax.bitcast_convert_type(lat, jnp.int32)
    key = jnp.where(ikey < 0, ikey ^ jnp.int32(0x7FFFFFFF), ikey)

    t = jnp.where(_count(key >= 0) >= K, jnp.int32(0), jnp.int32(INT_MIN))
    for bit in range(30, -1, -1):
        cand = t + jnp.int32(1 << bit)
        t = jnp.where(_count(key >= cand) >= K, cand, t)

    sparse = jnp.where(key >= t, lat, 0.0)
    sp_ref[...] = sparse

    rec = jax.lax.dot_general(
        sparse.astype(jnp.bfloat16), wd_ref[...], (((1,), (1,)), ((), ())),
        preferred_element_type=jnp.float32,
    ) + bd_ref[...]
    rec_ref[...] = rec


@jax.jit
def kernel(x, W_enc, b_enc, W_dec, b_dec):
    B = x.shape[0]
    grid = (B // TM,)
    out = pl.pallas_call(
        _body,
        grid=grid,
        in_specs=[
            pl.BlockSpec((TM, INPUT_DIM), lambda i: (i, 0)),
            pl.BlockSpec((LATENT_DIM, INPUT_DIM), lambda i: (0, 0)),
            pl.BlockSpec((1, LATENT_DIM), lambda i: (0, 0)),
            pl.BlockSpec((INPUT_DIM, LATENT_DIM), lambda i: (0, 0)),
            pl.BlockSpec((1, INPUT_DIM), lambda i: (0, 0)),
        ],
        out_specs=[
            pl.BlockSpec((TM, LATENT_DIM), lambda i: (i, 0)),
            pl.BlockSpec((TM, INPUT_DIM), lambda i: (i, 0)),
        ],
        out_shape=[
            jax.ShapeDtypeStruct((B, LATENT_DIM), jnp.float32),
            jax.ShapeDtypeStruct((B, INPUT_DIM), jnp.float32),
        ],
        compiler_params=pltpu.CompilerParams(
            vmem_limit_bytes=100 * 1024 * 1024,
        ),
    )(x, W_enc, b_enc.reshape(1, LATENT_DIM),
      W_dec.astype(jnp.bfloat16), b_dec.reshape(1, INPUT_DIM))
    sparse, recon = out
    return (recon, sparse)
